# baseline (speedup 1.0000x reference)
import functools

import jax
import jax.numpy as jnp
from jax import lax
from jax.experimental import pallas as pl
from jax.experimental.pallas import tpu as pltpu
from jax.experimental.pallas import tpu_sc as plsc

N = 10000
D = 128
DOUT = 40
DOUTP = 128
E = 320000
NP = 10240
NC = 2
NS = 16
NW = NC * NS
C = 128
CPT = 80
EPT = C * CPT
EP = EPT * NW
RPT = NP // NS
IB = 8
CPT0 = 152
CPT1 = 2 * CPT - CPT0


@functools.lru_cache(maxsize=None)
def _make_agg(rowdim):
  mesh = plsc.VectorSubcoreMesh(core_axis_name="c", subcore_axis_name="s")

  @functools.partial(
      pl.kernel,
      out_type=jax.ShapeDtypeStruct((NC, NP, rowdim), jnp.float32),
      mesh=mesh,
      scratch_types=[
          pltpu.VMEM((IB, C), jnp.int32),
          pltpu.VMEM((IB, C), jnp.int32),
          pltpu.VMEM((C, rowdim), jnp.float32),
          pltpu.VMEM((C, rowdim), jnp.float32),
          pltpu.VMEM_SHARED((NP, rowdim), jnp.float32),
          pltpu.SemaphoreType.DMA,
          pltpu.SemaphoreType.DMA,
      ],
      compiler_params=pltpu.CompilerParams(use_tc_tiling_on_sc=False),
  )
  def agg(table_hbm, srcs_hbm, dsts_hbm, zeros_hbm, out_hbm,
          idxs, idxd, rows0, rows1, accum, sem0, sem1):
    c = lax.axis_index("c")
    s = lax.axis_index("s")
    pltpu.sync_copy(zeros_hbm, accum.at[pl.ds(s * RPT, RPT)])
    plsc.subcore_barrier()

    my_base = lax.select(c == 0, s * CPT0, NS * CPT0 + s * CPT1)
    my_halves = lax.select(c == 0, CPT0 // IB, CPT1 // IB)

    def half_body(h, carry):
      base = my_base + h * IB
      pltpu.sync_copy(srcs_hbm.at[pl.ds(base, IB)], idxs)
      pltpu.sync_copy(dsts_hbm.at[pl.ds(base, IB)], idxd)
      pltpu.async_copy(table_hbm.at[idxs.at[0]], rows0, sem0)

      def body(k, carry2):
        j = 2 * k
        pltpu.make_async_copy(table_hbm.at[idxs.at[j]], rows0, sem0).wait()
        pltpu.async_copy(table_hbm.at[idxs.at[j + 1]], rows1, sem1)
        pltpu.sync_copy(rows0, accum.at[idxd.at[j]], add=True)
        pltpu.make_async_copy(table_hbm.at[idxs.at[j]], rows1, sem1).wait()

        @pl.when(k < IB // 2 - 1)
        def _():
          pltpu.async_copy(table_hbm.at[idxs.at[j + 2]], rows0, sem0)

        pltpu.sync_copy(rows1, accum.at[idxd.at[j + 1]], add=True)
        return carry2

      lax.fori_loop(0, IB // 2, body, 0)
      return carry

    lax.fori_loop(0, my_halves, half_body, 0)
    plsc.subcore_barrier()
    pltpu.sync_copy(accum.at[pl.ds(s * RPT, RPT)],
                    out_hbm.at[c, pl.ds(s * RPT, RPT)])

  return agg


@functools.lru_cache(maxsize=None)
def _make_deg():
  mesh = plsc.VectorSubcoreMesh(core_axis_name="c", subcore_axis_name="s")

  @functools.partial(
      pl.kernel,
      out_type=jax.ShapeDtypeStruct((NC, NP, 16), jnp.float32),
      mesh=mesh,
      scratch_types=[
          pltpu.VMEM((CPT, C), jnp.int32),
          pltpu.VMEM((C, 16), jnp.float32),
          pltpu.VMEM_SHARED((NP, 16), jnp.float32),
      ],
      compiler_params=pltpu.CompilerParams(use_tc_tiling_on_sc=False),
  )
  def deg(dsts_hbm, ones_hbm, zeros_hbm, out_hbm, idxd, ones_b, accum):
    c = lax.axis_index("c")
    s = lax.axis_index("s")
    w = s * NC + c
    pltpu.sync_copy(dsts_hbm.at[pl.ds(w * CPT, CPT)], idxd)
    pltpu.sync_copy(ones_hbm, ones_b)
    pltpu.sync_copy(zeros_hbm, accum.at[pl.ds(s * RPT, RPT)])
    plsc.subcore_barrier()

    def body(j, carry):
      pltpu.sync_copy(ones_b, accum.at[idxd.at[j]], add=True)
      return carry

    lax.fori_loop(0, CPT, body, 0)
    plsc.subcore_barrier()
    pltpu.sync_copy(accum.at[pl.ds(s * RPT, RPT)],
                    out_hbm.at[c, pl.ds(s * RPT, RPT)])

  return deg


def _dinv_col(dp_ref):
  deg = 1.0 + dp_ref[0, :, 0:1] + dp_ref[1, :, 0:1]
  return lax.rsqrt(deg)


def _first_body(x_ref, w_ref, dp_ref, o_ref):
  dinv = _dinv_col(dp_ref)
  y = jnp.dot(x_ref[...], w_ref[...], preferred_element_type=jnp.float32)
  o_ref[...] = y * dinv


def _tc_first(x_p, W1, deg_parts):
  return pl.pallas_call(
      _first_body,
      out_shape=jax.ShapeDtypeStruct((NP, D), jnp.float32),
  )(x_p, W1, deg_parts)


def _mid_body(p_ref, y_ref, dp_ref, b_ref, g_ref, be_ref, w_ref, o_ref):
  dinv = _dinv_col(dp_ref)
  h = dinv * (p_ref[0] + p_ref[1] + y_ref[...]) + b_ref[...]
  mask = lax.broadcasted_iota(jnp.int32, (NP, 1), 0) < N
  h = jnp.where(mask, h, 0.0)
  mu = jnp.sum(h, axis=0, keepdims=True) * (1.0 / N)
  d = h - mu
  var = jnp.sum(jnp.where(mask, d * d, 0.0), axis=0, keepdims=True) * (1.0 / N)
  hn = d * lax.rsqrt(var + 1e-5) * g_ref[...] + be_ref[...]
  hn = jnp.maximum(hn, 0.0)
  hn = jnp.where(mask, hn, 0.0)
  o_ref[...] = jnp.dot(hn, w_ref[...], preferred_element_type=jnp.float32) * dinv


def _tc_mid(p, y, deg_parts, b, g, be, Wn, outdim):
  return pl.pallas_call(
      _mid_body,
      out_shape=jax.ShapeDtypeStruct((NP, outdim), jnp.float32),
  )(p, y, deg_parts, b, g, be, Wn)


def _fin_body(p_ref, y_ref, dp_ref, b_ref, o_ref):
  dinv = _dinv_col(dp_ref)
  h = dinv * (p_ref[0] + p_ref[1] + y_ref[...]) + b_ref[...]
  z = h[:N, :DOUT]
  m = jnp.max(z, axis=1, keepdims=True)
  zz = z - m
  lse = jnp.log(jnp.sum(jnp.exp(zz), axis=1, keepdims=True))
  o_ref[...] = zz - lse


def _tc_fin(p, y, deg_parts, b):
  return pl.pallas_call(
      _fin_body,
      out_shape=jax.ShapeDtypeStruct((N, DOUT), jnp.float32),
  )(p, y, deg_parts, b)


def kernel(x, edge_index, W1, b1, g1, be1, W2, b2, g2, be2, W3, b3):
  src = edge_index[0]
  dst = edge_index[1]
  pad_idx = jnp.full((EP - E,), NP - 1, jnp.int32)
  srcs = jnp.concatenate([src, pad_idx]).reshape(EP // C, C)
  dsts = jnp.concatenate([dst, pad_idx]).reshape(EP // C, C)
  x_p = jnp.zeros((NP, D), jnp.float32).at[:N].set(x)

  zeros16 = jnp.zeros((RPT, 16), jnp.float32)
  zeros128 = jnp.zeros((RPT, D), jnp.float32)
  ones16 = jnp.ones((C, 16), jnp.float32)
  W3p = jnp.zeros((D, DOUTP), jnp.float32).at[:, :DOUT].set(W3)
  b1r = b1.reshape(1, D)
  g1r = g1.reshape(1, D)
  be1r = be1.reshape(1, D)
  b2r = b2.reshape(1, D)
  g2r = g2.reshape(1, D)
  be2r = be2.reshape(1, D)
  b3r = jnp.zeros((1, DOUTP), jnp.float32).at[0, :DOUT].set(b3)

  deg_parts = _make_deg()(dsts, ones16, zeros16)
  y1 = _tc_first(x_p, W1, deg_parts)
  p1 = _make_agg(D)(y1, srcs, dsts, zeros128)
  y2 = _tc_mid(p1, y1, deg_parts, b1r, g1r, be1r, W2, D)
  p2 = _make_agg(D)(y2, srcs, dsts, zeros128)
  y3 = _tc_mid(p2, y2, deg_parts, b2r, g2r, be2r, W3p, DOUTP)
  p3 = _make_agg(DOUTP)(y3, srcs, dsts, zeros128)
  return _tc_fin(p3, y3, deg_parts, b3r)


# speedup vs baseline: 1.0529x; 1.0529x over previous
import functools

import jax
import jax.numpy as jnp
from jax import lax
from jax.experimental import pallas as pl
from jax.experimental.pallas import tpu as pltpu
from jax.experimental.pallas import tpu_sc as plsc

N = 10000
D = 128
DOUT = 40
DOUTP = 128
E = 320000
NP = 10240
NC = 2
NS = 16
NW = NC * NS
C = 128
CPT = 80
EPT = C * CPT
EP = EPT * NW
RPT = NP // NS
IB = 8
CPT0 = 144
CPT1 = 2 * CPT - CPT0


@functools.lru_cache(maxsize=None)
def _make_agg(rowdim):
  mesh = plsc.VectorSubcoreMesh(core_axis_name="c", subcore_axis_name="s")

  @functools.partial(
      pl.kernel,
      out_type=jax.ShapeDtypeStruct((NC, NP, rowdim), jnp.float32),
      mesh=mesh,
      scratch_types=[
          pltpu.VMEM((IB, C), jnp.int32),
          pltpu.VMEM((IB, C), jnp.int32),
          pltpu.VMEM((C, rowdim), jnp.float32),
          pltpu.VMEM((C, rowdim), jnp.float32),
          pltpu.VMEM_SHARED((NP, rowdim), jnp.float32),
          pltpu.SemaphoreType.DMA,
          pltpu.SemaphoreType.DMA,
      ],
      compiler_params=pltpu.CompilerParams(use_tc_tiling_on_sc=False),
  )
  def agg(table_hbm, srcs_hbm, dsts_hbm, zeros_hbm, out_hbm,
          idxs, idxd, rows0, rows1, accum, sem0, sem1):
    c = lax.axis_index("c")
    s = lax.axis_index("s")
    pltpu.sync_copy(zeros_hbm, accum.at[pl.ds(s * RPT, RPT)])
    plsc.subcore_barrier()

    my_base = lax.select(c == 0, s * CPT0, NS * CPT0 + s * CPT1)
    my_halves = lax.select(c == 0, CPT0 // IB, CPT1 // IB)

    def half_body(h, carry):
      base = my_base + h * IB
      pltpu.sync_copy(srcs_hbm.at[pl.ds(base, IB)], idxs)
      pltpu.sync_copy(dsts_hbm.at[pl.ds(base, IB)], idxd)
      pltpu.async_copy(table_hbm.at[idxs.at[0]], rows0, sem0)

      def body(k, carry2):
        j = 2 * k
        pltpu.make_async_copy(table_hbm.at[idxs.at[j]], rows0, sem0).wait()
        pltpu.async_copy(table_hbm.at[idxs.at[j + 1]], rows1, sem1)
        pltpu.sync_copy(rows0, accum.at[idxd.at[j]], add=True)
        pltpu.make_async_copy(table_hbm.at[idxs.at[j]], rows1, sem1).wait()

        @pl.when(k < IB // 2 - 1)
        def _():
          pltpu.async_copy(table_hbm.at[idxs.at[j + 2]], rows0, sem0)

        pltpu.sync_copy(rows1, accum.at[idxd.at[j + 1]], add=True)
        return carry2

      lax.fori_loop(0, IB // 2, body, 0)
      return carry

    lax.fori_loop(0, my_halves, half_body, 0)
    plsc.subcore_barrier()
    pltpu.sync_copy(accum.at[pl.ds(s * RPT, RPT)],
                    out_hbm.at[c, pl.ds(s * RPT, RPT)])

  return agg


@functools.lru_cache(maxsize=None)
def _make_deg():
  mesh = plsc.VectorSubcoreMesh(core_axis_name="c", subcore_axis_name="s")

  @functools.partial(
      pl.kernel,
      out_type=jax.ShapeDtypeStruct((NC, NP, 16), jnp.float32),
      mesh=mesh,
      scratch_types=[
          pltpu.VMEM((CPT, C), jnp.int32),
          pltpu.VMEM((C, 16), jnp.float32),
          pltpu.VMEM_SHARED((NP, 16), jnp.float32),
      ],
      compiler_params=pltpu.CompilerParams(use_tc_tiling_on_sc=False),
  )
  def deg(dsts_hbm, ones_hbm, zeros_hbm, out_hbm, idxd, ones_b, accum):
    c = lax.axis_index("c")
    s = lax.axis_index("s")
    w = s * NC + c
    pltpu.sync_copy(dsts_hbm.at[pl.ds(w * CPT, CPT)], idxd)
    pltpu.sync_copy(ones_hbm, ones_b)
    pltpu.sync_copy(zeros_hbm, accum.at[pl.ds(s * RPT, RPT)])
    plsc.subcore_barrier()

    def body(j, carry):
      pltpu.sync_copy(ones_b, accum.at[idxd.at[j]], add=True)
      return carry

    lax.fori_loop(0, CPT, body, 0)
    plsc.subcore_barrier()
    pltpu.sync_copy(accum.at[pl.ds(s * RPT, RPT)],
                    out_hbm.at[c, pl.ds(s * RPT, RPT)])

  return deg


def _dinv_col(dp_ref):
  deg = 1.0 + dp_ref[0, :, 0:1] + dp_ref[1, :, 0:1]
  return lax.rsqrt(deg)


def _first_body(x_ref, w_ref, dp_ref, o_ref):
  dinv = _dinv_col(dp_ref)
  y = jnp.dot(x_ref[...], w_ref[...], preferred_element_type=jnp.float32)
  o_ref[...] = y * dinv


def _tc_first(x_p, W1, deg_parts):
  return pl.pallas_call(
      _first_body,
      out_shape=jax.ShapeDtypeStruct((NP, D), jnp.float32),
  )(x_p, W1, deg_parts)


def _mid_body(p_ref, y_ref, dp_ref, b_ref, g_ref, be_ref, w_ref, o_ref):
  dinv = _dinv_col(dp_ref)
  h = dinv * (p_ref[0] + p_ref[1] + y_ref[...]) + b_ref[...]
  mask = lax.broadcasted_iota(jnp.int32, (NP, 1), 0) < N
  h = jnp.where(mask, h, 0.0)
  mu = jnp.sum(h, axis=0, keepdims=True) * (1.0 / N)
  d = h - mu
  var = jnp.sum(jnp.where(mask, d * d, 0.0), axis=0, keepdims=True) * (1.0 / N)
  hn = d * lax.rsqrt(var + 1e-5) * g_ref[...] + be_ref[...]
  hn = jnp.maximum(hn, 0.0)
  hn = jnp.where(mask, hn, 0.0)
  o_ref[...] = jnp.dot(hn, w_ref[...], preferred_element_type=jnp.float32) * dinv


def _tc_mid(p, y, deg_parts, b, g, be, Wn, outdim):
  return pl.pallas_call(
      _mid_body,
      out_shape=jax.ShapeDtypeStruct((NP, outdim), jnp.float32),
  )(p, y, deg_parts, b, g, be, Wn)


def _fin_body(p_ref, y_ref, dp_ref, b_ref, o_ref):
  dinv = _dinv_col(dp_ref)
  h = dinv * (p_ref[0] + p_ref[1] + y_ref[...]) + b_ref[...]
  z = h[:N, :DOUT]
  m = jnp.max(z, axis=1, keepdims=True)
  zz = z - m
  lse = jnp.log(jnp.sum(jnp.exp(zz), axis=1, keepdims=True))
  o_ref[...] = zz - lse


def _tc_fin(p, y, deg_parts, b):
  return pl.pallas_call(
      _fin_body,
      out_shape=jax.ShapeDtypeStruct((N, DOUT), jnp.float32),
  )(p, y, deg_parts, b)


def kernel(x, edge_index, W1, b1, g1, be1, W2, b2, g2, be2, W3, b3):
  src = edge_index[0]
  dst = edge_index[1]
  pad_idx = jnp.full((EP - E,), NP - 1, jnp.int32)
  srcs = jnp.concatenate([src, pad_idx]).reshape(EP // C, C)
  dsts = jnp.concatenate([dst, pad_idx]).reshape(EP // C, C)
  x_p = jnp.zeros((NP, D), jnp.float32).at[:N].set(x)

  zeros16 = jnp.zeros((RPT, 16), jnp.float32)
  zeros128 = jnp.zeros((RPT, D), jnp.float32)
  ones16 = jnp.ones((C, 16), jnp.float32)
  W3p = jnp.zeros((D, DOUTP), jnp.float32).at[:, :DOUT].set(W3)
  b1r = b1.reshape(1, D)
  g1r = g1.reshape(1, D)
  be1r = be1.reshape(1, D)
  b2r = b2.reshape(1, D)
  g2r = g2.reshape(1, D)
  be2r = be2.reshape(1, D)
  b3r = jnp.zeros((1, DOUTP), jnp.float32).at[0, :DOUT].set(b3)

  deg_parts = _make_deg()(dsts, ones16, zeros16)
  y1 = _tc_first(x_p, W1, deg_parts)
  p1 = _make_agg(D)(y1, srcs, dsts, zeros128)
  y2 = _tc_mid(p1, y1, deg_parts, b1r, g1r, be1r, W2, D)
  p2 = _make_agg(D)(y2, srcs, dsts, zeros128)
  y3 = _tc_mid(p2, y2, deg_parts, b2r, g2r, be2r, W3p, DOUTP)
  p3 = _make_agg(DOUTP)(y3, srcs, dsts, zeros128)
  return _tc_fin(p3, y3, deg_parts, b3r)


# speedup vs baseline: 1.0530x; 1.0001x over previous
import functools

import jax
import jax.numpy as jnp
from jax import lax
from jax.experimental import pallas as pl
from jax.experimental.pallas import tpu as pltpu
from jax.experimental.pallas import tpu_sc as plsc

N = 10000
D = 128
DOUT = 40
DOUTP = 128
E = 320000
NP = 10240
NC = 2
NS = 16
NW = NC * NS
C = 128
CPT = 80
EPT = C * CPT
EP = EPT * NW
RPT = NP // NS
IB = 8
CPT0 = 144
CPT1 = 2 * CPT - CPT0


@functools.lru_cache(maxsize=None)
def _make_agg(rowdim):
  mesh = plsc.VectorSubcoreMesh(core_axis_name="c", subcore_axis_name="s")

  @functools.partial(
      pl.kernel,
      out_type=jax.ShapeDtypeStruct((NC, NP, rowdim), jnp.float32),
      mesh=mesh,
      scratch_types=[
          pltpu.VMEM((IB, C), jnp.int32),
          pltpu.VMEM((IB, C), jnp.int32),
          pltpu.VMEM((C, rowdim), jnp.float32),
          pltpu.VMEM((C, rowdim), jnp.float32),
          pltpu.VMEM_SHARED((NP, rowdim), jnp.float32),
          pltpu.SemaphoreType.DMA,
          pltpu.SemaphoreType.DMA,
          pltpu.SemaphoreType.DMA,
          pltpu.SemaphoreType.DMA,
      ],
      compiler_params=pltpu.CompilerParams(use_tc_tiling_on_sc=False),
  )
  def agg(table_hbm, srcs_hbm, dsts_hbm, zeros_hbm, out_hbm,
          idxs, idxd, rows0, rows1, accum, sem0, sem1, ssem0, ssem1):
    c = lax.axis_index("c")
    s = lax.axis_index("s")
    pltpu.sync_copy(zeros_hbm, accum.at[pl.ds(s * RPT, RPT)])
    plsc.subcore_barrier()

    my_base = lax.select(c == 0, s * CPT0, NS * CPT0 + s * CPT1)
    my_halves = lax.select(c == 0, CPT0 // IB, CPT1 // IB)

    def half_body(h, carry):
      base = my_base + h * IB
      pltpu.sync_copy(srcs_hbm.at[pl.ds(base, IB)], idxs)
      pltpu.sync_copy(dsts_hbm.at[pl.ds(base, IB)], idxd)
      pltpu.async_copy(table_hbm.at[idxs.at[0]], rows0, sem0)
      pltpu.async_copy(table_hbm.at[idxs.at[1]], rows1, sem1)

      def body(k, carry2):
        j = 2 * k
        pltpu.make_async_copy(table_hbm.at[idxs.at[j]], rows0, sem0).wait()
        pltpu.async_copy(rows0, accum.at[idxd.at[j]], ssem0, add=True)
        pltpu.make_async_copy(table_hbm.at[idxs.at[j]], rows1, sem1).wait()
        pltpu.async_copy(rows1, accum.at[idxd.at[j + 1]], ssem1, add=True)

        @pl.when(k < IB // 2 - 1)
        def _():
          pltpu.make_async_copy(rows0, accum.at[idxd.at[j]], ssem0).wait()
          pltpu.async_copy(table_hbm.at[idxs.at[j + 2]], rows0, sem0)
          pltpu.make_async_copy(rows1, accum.at[idxd.at[j]], ssem1).wait()
          pltpu.async_copy(table_hbm.at[idxs.at[j + 3]], rows1, sem1)

        return carry2

      lax.fori_loop(0, IB // 2, body, 0)
      pltpu.make_async_copy(rows0, accum.at[idxd.at[0]], ssem0).wait()
      pltpu.make_async_copy(rows1, accum.at[idxd.at[0]], ssem1).wait()
      return carry

    lax.fori_loop(0, my_halves, half_body, 0)
    plsc.subcore_barrier()
    pltpu.sync_copy(accum.at[pl.ds(s * RPT, RPT)],
                    out_hbm.at[c, pl.ds(s * RPT, RPT)])

  return agg


@functools.lru_cache(maxsize=None)
def _make_deg():
  mesh = plsc.VectorSubcoreMesh(core_axis_name="c", subcore_axis_name="s")

  @functools.partial(
      pl.kernel,
      out_type=jax.ShapeDtypeStruct((NC, NP, 16), jnp.float32),
      mesh=mesh,
      scratch_types=[
          pltpu.VMEM((CPT, C), jnp.int32),
          pltpu.VMEM((C, 16), jnp.float32),
          pltpu.VMEM_SHARED((NP, 16), jnp.float32),
      ],
      compiler_params=pltpu.CompilerParams(use_tc_tiling_on_sc=False),
  )
  def deg(dsts_hbm, ones_hbm, zeros_hbm, out_hbm, idxd, ones_b, accum):
    c = lax.axis_index("c")
    s = lax.axis_index("s")
    w = s * NC + c
    pltpu.sync_copy(dsts_hbm.at[pl.ds(w * CPT, CPT)], idxd)
    pltpu.sync_copy(ones_hbm, ones_b)
    pltpu.sync_copy(zeros_hbm, accum.at[pl.ds(s * RPT, RPT)])
    plsc.subcore_barrier()

    def body(j, carry):
      pltpu.sync_copy(ones_b, accum.at[idxd.at[j]], add=True)
      return carry

    lax.fori_loop(0, CPT, body, 0)
    plsc.subcore_barrier()
    pltpu.sync_copy(accum.at[pl.ds(s * RPT, RPT)],
                    out_hbm.at[c, pl.ds(s * RPT, RPT)])

  return deg


def _dinv_col(dp_ref):
  deg = 1.0 + dp_ref[0, :, 0:1] + dp_ref[1, :, 0:1]
  return lax.rsqrt(deg)


def _first_body(x_ref, w_ref, dp_ref, o_ref):
  dinv = _dinv_col(dp_ref)
  y = jnp.dot(x_ref[...], w_ref[...], preferred_element_type=jnp.float32)
  o_ref[...] = y * dinv


def _tc_first(x_p, W1, deg_parts):
  return pl.pallas_call(
      _first_body,
      out_shape=jax.ShapeDtypeStruct((NP, D), jnp.float32),
  )(x_p, W1, deg_parts)


def _mid_body(p_ref, y_ref, dp_ref, b_ref, g_ref, be_ref, w_ref, o_ref):
  dinv = _dinv_col(dp_ref)
  h = dinv * (p_ref[0] + p_ref[1] + y_ref[...]) + b_ref[...]
  mask = lax.broadcasted_iota(jnp.int32, (NP, 1), 0) < N
  h = jnp.where(mask, h, 0.0)
  mu = jnp.sum(h, axis=0, keepdims=True) * (1.0 / N)
  d = h - mu
  var = jnp.sum(jnp.where(mask, d * d, 0.0), axis=0, keepdims=True) * (1.0 / N)
  hn = d * lax.rsqrt(var + 1e-5) * g_ref[...] + be_ref[...]
  hn = jnp.maximum(hn, 0.0)
  hn = jnp.where(mask, hn, 0.0)
  o_ref[...] = jnp.dot(hn, w_ref[...], preferred_element_type=jnp.float32) * dinv


def _tc_mid(p, y, deg_parts, b, g, be, Wn, outdim):
  return pl.pallas_call(
      _mid_body,
      out_shape=jax.ShapeDtypeStruct((NP, outdim), jnp.float32),
  )(p, y, deg_parts, b, g, be, Wn)


def _fin_body(p_ref, y_ref, dp_ref, b_ref, o_ref):
  dinv = _dinv_col(dp_ref)
  h = dinv * (p_ref[0] + p_ref[1] + y_ref[...]) + b_ref[...]
  z = h[:N, :DOUT]
  m = jnp.max(z, axis=1, keepdims=True)
  zz = z - m
  lse = jnp.log(jnp.sum(jnp.exp(zz), axis=1, keepdims=True))
  o_ref[...] = zz - lse


def _tc_fin(p, y, deg_parts, b):
  return pl.pallas_call(
      _fin_body,
      out_shape=jax.ShapeDtypeStruct((N, DOUT), jnp.float32),
  )(p, y, deg_parts, b)


def kernel(x, edge_index, W1, b1, g1, be1, W2, b2, g2, be2, W3, b3):
  src = edge_index[0]
  dst = edge_index[1]
  pad_idx = jnp.full((EP - E,), NP - 1, jnp.int32)
  srcs = jnp.concatenate([src, pad_idx]).reshape(EP // C, C)
  dsts = jnp.concatenate([dst, pad_idx]).reshape(EP // C, C)
  x_p = jnp.zeros((NP, D), jnp.float32).at[:N].set(x)

  zeros16 = jnp.zeros((RPT, 16), jnp.float32)
  zeros128 = jnp.zeros((RPT, D), jnp.float32)
  ones16 = jnp.ones((C, 16), jnp.float32)
  W3p = jnp.zeros((D, DOUTP), jnp.float32).at[:, :DOUT].set(W3)
  b1r = b1.reshape(1, D)
  g1r = g1.reshape(1, D)
  be1r = be1.reshape(1, D)
  b2r = b2.reshape(1, D)
  g2r = g2.reshape(1, D)
  be2r = be2.reshape(1, D)
  b3r = jnp.zeros((1, DOUTP), jnp.float32).at[0, :DOUT].set(b3)

  deg_parts = _make_deg()(dsts, ones16, zeros16)
  y1 = _tc_first(x_p, W1, deg_parts)
  p1 = _make_agg(D)(y1, srcs, dsts, zeros128)
  y2 = _tc_mid(p1, y1, deg_parts, b1r, g1r, be1r, W2, D)
  p2 = _make_agg(D)(y2, srcs, dsts, zeros128)
  y3 = _tc_mid(p2, y2, deg_parts, b2r, g2r, be2r, W3p, DOUTP)
  p3 = _make_agg(DOUTP)(y3, srcs, dsts, zeros128)
  return _tc_fin(p3, y3, deg_parts, b3r)
